# baseline (speedup 1.0000x reference)
import jax
import jax.numpy as jnp
from jax import lax
from jax.experimental import pallas as pl
from jax.experimental.pallas import tpu as pltpu
from jax.experimental.pallas import tpu_sc as plsc

NC = 2
NS = 16
NW = NC * NS
L = 16

C = 1280
NSETS = 4
PROWS = 4


def _sc_kernel_fn(n, n_pad, t_edges, a_anchors, tailp):
  nsl = n_pad // NS
  e_t = (t_edges // (NW * 128)) * 128
  nfull = e_t // C
  rem = e_t - nfull * C
  a_s = a_anchors // NS
  a_rows = a_s // 128
  n_tbl = n_pad

  f32 = jnp.float32

  pipelined = (
      nfull >= 2 and (nfull - 2) % NSETS == 0
      and (NW - 1) * e_t + (nfull + 1) * C + C <= t_edges
  )

  nchunks = []
  off = 0
  while off < nsl:
    nchunks.append((off, min(C, nsl - off)))
    off += C

  def body(x_hbm, w_hbm, src_hbm, dst_hbm, anc_hbm, tsrc_hbm, tdst_hbm, tw_hbm,
           s_out, c_out, h_out,
           h_table,
           src_0, src_1, src_2, src_3,
           w_0, w_1, w_2, w_3,
           di_0, di_1, di_2, di_3,
           pstage, ones_v,
           acc_s, acc_c,
           sem_0, sem_1, sem_2, sem_3, sem_s):
    c = lax.axis_index("c")
    s = lax.axis_index("s")
    wid = c * NS + s
    nb = s * nsl

    sets = [
        (src_0, w_0, di_0, sem_0),
        (src_1, w_1, di_1, sem_1),
        (src_2, w_2, di_2, sem_2),
        (src_3, w_3, di_3, sem_3),
    ]

    def init_ones(i, _):
      ones_v[pl.ds(i * L, L)] = jnp.ones((L,), f32)
      return 0
    lax.fori_loop(0, C // L, init_ones, 0)

    def zero_w2(i, _):
      w_2[pl.ds(i * L, L)] = jnp.zeros((L,), f32)
      return 0
    lax.fori_loop(0, C // L, zero_w2, 0)

    for arr in (acc_s, acc_c):
      for noff, nsz in nchunks:
        pltpu.async_copy(w_2.at[pl.ds(0, nsz)], arr.at[pl.ds(nb + noff, nsz)],
                         sem_s)
    for arr in (acc_s, acc_c):
      for noff, nsz in nchunks:
        pltpu.make_async_copy(w_2.at[pl.ds(0, nsz)],
                              arr.at[pl.ds(nb + noff, nsz)], sem_s).wait()
    plsc.subcore_barrier()

    for r in range(a_rows):
      pltpu.sync_copy(anc_hbm.at[pl.ds(s * a_s + r * 128, 128)], pstage.at[r])
    for r in range(a_rows):
      pltpu.async_copy(x_hbm.at[pstage.at[r]], w_3.at[pl.ds(r * 128, 128)],
                       sem_s)
    for r in range(a_rows):
      pltpu.make_async_copy(x_hbm.at[pstage.at[r]],
                            w_3.at[pl.ds(r * 128, 128)], sem_s).wait()
    for r in range(a_rows):
      pltpu.async_copy(w_3.at[pl.ds(r * 128, 128)], acc_s.at[pstage.at[r]],
                       sem_s, add=True)
      pltpu.async_copy(ones_v.at[pl.ds(0, 128)], acc_c.at[pstage.at[r]],
                       sem_s, add=True)
    for r in range(a_rows):
      pltpu.make_async_copy(w_3.at[pl.ds(r * 128, 128)],
                            acc_s.at[pstage.at[r]], sem_s).wait()
      pltpu.make_async_copy(ones_v.at[pl.ds(0, 128)],
                            acc_c.at[pstage.at[r]], sem_s).wait()
    plsc.subcore_barrier()

    for noff, nsz in nchunks:
      pltpu.async_copy(acc_c.at[pl.ds(nb + noff, nsz)], w_0.at[pl.ds(0, nsz)],
                       sem_s)
      pltpu.async_copy(acc_s.at[pl.ds(nb + noff, nsz)], w_1.at[pl.ds(0, nsz)],
                       sem_s)
      pltpu.make_async_copy(acc_c.at[pl.ds(nb + noff, nsz)],
                            w_0.at[pl.ds(0, nsz)], sem_s).wait()
      pltpu.make_async_copy(acc_s.at[pl.ds(nb + noff, nsz)],
                            w_1.at[pl.ds(0, nsz)], sem_s).wait()
      for arr in (acc_s, acc_c):
        pltpu.async_copy(w_2.at[pl.ds(0, nsz)], arr.at[pl.ds(nb + noff, nsz)],
                         sem_s)

      def hbody(i, _):
        hcv = w_0[pl.ds(i * L, L)]
        hgv = w_1[pl.ds(i * L, L)]
        w_1[pl.ds(i * L, L)] = jnp.where(hcv > 0.0, hgv + 1.0,
                                         jnp.zeros((L,), f32))
        return 0
      lax.fori_loop(0, nsz // L, hbody, 0)
      pltpu.sync_copy(w_1.at[pl.ds(0, nsz)], h_out.at[c, pl.ds(nb + noff, nsz)])

    for arr in (acc_s, acc_c):
      for noff, nsz in nchunks:
        pltpu.make_async_copy(w_2.at[pl.ds(0, nsz)],
                              arr.at[pl.ds(nb + noff, nsz)], sem_s).wait()
    plsc.subcore_barrier()

    pltpu.sync_copy(h_out.at[c], h_table)

    tbase = wid * e_t

    def start_loads(b, st):
      sbuf, wbuf, dbuf, sem = st
      pltpu.async_copy(src_hbm.at[pl.ds(b, C)], sbuf, sem)
      pltpu.async_copy(w_hbm.at[pl.ds(b, C)], wbuf, sem)
      pltpu.async_copy(dst_hbm.at[pl.ds(b, C)], dbuf, sem)

    def wait_loads(b, st):
      sbuf, wbuf, dbuf, sem = st
      pltpu.make_async_copy(src_hbm.at[pl.ds(b, C)], sbuf, sem).wait()
      pltpu.make_async_copy(w_hbm.at[pl.ds(b, C)], wbuf, sem).wait()
      pltpu.make_async_copy(dst_hbm.at[pl.ds(b, C)], dbuf, sem).wait()

    def compute(st, cs):
      sbuf, wbuf, _, _ = st

      def grp(i, _):
        for u in range(4):
          o = i * 4 * L + u * L
          sv = sbuf[pl.ds(o, L)]
          hv = plsc.load_gather(h_table, [sv])
          wv = wbuf[pl.ds(o, L)]
          wbuf[pl.ds(o, L)] = hv * wv
        return 0
      lax.fori_loop(0, cs // (4 * L), grp, 0)

    def fire(st):
      _, wbuf, dbuf, sem = st
      pltpu.async_copy(wbuf, acc_s.at[dbuf], sem, add=True)
      pltpu.async_copy(ones_v, acc_c.at[dbuf], sem, add=True)

    def drain(st):
      _, wbuf, dbuf, sem = st
      pltpu.make_async_copy(wbuf, acc_s.at[dbuf], sem).wait()
      pltpu.make_async_copy(ones_v, acc_c.at[dbuf], sem).wait()

    if pipelined:
      start_loads(tbase, sets[0])
      start_loads(tbase + C, sets[1])
      start_loads(tbase + 2 * C, sets[2])
      wait_loads(tbase, sets[0])
      compute(sets[0], C)
      fire(sets[0])
      start_loads(tbase + 3 * C, sets[3])
      wait_loads(tbase + C, sets[1])
      compute(sets[1], C)
      fire(sets[1])

      def quad(k4, _):
        for u in range(NSETS):
          k = 2 + u
          b = tbase + (4 * k4 + k) * C
          P = sets[k % NSETS]
          SD = sets[u]
          drain(SD)
          start_loads(b + 2 * C, SD)
          wait_loads(b, P)
          compute(P, C)
          fire(P)
        return 0
      lax.fori_loop(0, (nfull - 2) // NSETS, quad, 0)

      drain(sets[(nfull - 2) % NSETS])
      drain(sets[(nfull - 1) % NSETS])
      wait_loads(tbase + nfull * C, sets[nfull % NSETS])
      wait_loads(tbase + (nfull + 1) * C, sets[(nfull + 1) % NSETS])
      done = nfull * C
    else:
      done = 0

    def chunk_sync(sref, dref, wref, b, cs):
      pltpu.sync_copy(sref.at[pl.ds(b, cs)], src_0.at[pl.ds(0, cs)])
      pltpu.sync_copy(wref.at[pl.ds(b, cs)], w_0.at[pl.ds(0, cs)])
      pltpu.sync_copy(dref.at[pl.ds(b, cs)], di_0.at[pl.ds(0, cs)])
      if cs == C:
        compute(sets[0], C)
        fire(sets[0])
        drain(sets[0])
      else:
        nrows = cs // 128

        def row(j, _):
          for k in range(128 // L):
            o = j * 128 + k * L
            sv = src_0[pl.ds(o, L)]
            hv = plsc.load_gather(h_table, [sv])
            wv = w_0[pl.ds(o, L)]
            w_0[pl.ds(o, L)] = hv * wv
            pstage[j, pl.ds(k * L, L)] = di_0[pl.ds(o, L)]
          return 0
        lax.fori_loop(0, nrows, row, 0)

        def fire_r(j, _):
          pltpu.async_copy(w_0.at[pl.ds(j * 128, 128)],
                           acc_s.at[pstage.at[j]], sem_s, add=True)
          pltpu.async_copy(ones_v.at[pl.ds(0, 128)],
                           acc_c.at[pstage.at[j]], sem_s, add=True)
          return 0
        lax.fori_loop(0, nrows, fire_r, 0)

        def drain_r(j, _):
          pltpu.make_async_copy(w_0.at[pl.ds(j * 128, 128)],
                                acc_s.at[pstage.at[j]], sem_s).wait()
          pltpu.make_async_copy(ones_v.at[pl.ds(0, 128)],
                                acc_c.at[pstage.at[j]], sem_s).wait()
          return 0
        lax.fori_loop(0, nrows, drain_r, 0)

    def piece_sizes(total):
      sizes = []
      left = total
      while left > 0:
        cs = min(C, left)
        if cs < C:
          cs = min(PROWS * 128, cs)
        sizes.append(cs)
        left -= cs
      return sizes

    off = done
    for cs in piece_sizes(e_t - done):
      chunk_sync(src_hbm, dst_hbm, w_hbm, tbase + off, cs)
      off += cs

    if tailp:
      trows = tailp // 128
      for q in range((trows + NW - 1) // NW):
        rid = wid + NW * q

        @pl.when(rid < trows)
        def _():
          chunk_sync(tsrc_hbm, tdst_hbm, tw_hbm, rid * 128, 128)

    plsc.subcore_barrier()

    for noff, nsz in nchunks:
      pltpu.async_copy(acc_s.at[pl.ds(nb + noff, nsz)],
                       s_out.at[c, pl.ds(nb + noff, nsz)], sem_s)
      pltpu.async_copy(acc_c.at[pl.ds(nb + noff, nsz)],
                       c_out.at[c, pl.ds(nb + noff, nsz)], sem_s)
    for noff, nsz in nchunks:
      pltpu.make_async_copy(acc_s.at[pl.ds(nb + noff, nsz)],
                            s_out.at[c, pl.ds(nb + noff, nsz)], sem_s).wait()
      pltpu.make_async_copy(acc_c.at[pl.ds(nb + noff, nsz)],
                            c_out.at[c, pl.ds(nb + noff, nsz)], sem_s).wait()

  i32 = jnp.int32
  return pl.kernel(
      body,
      out_type=(
          jax.ShapeDtypeStruct((NC, n_pad), f32),
          jax.ShapeDtypeStruct((NC, n_pad), f32),
          jax.ShapeDtypeStruct((NC, n_pad), f32),
      ),
      mesh=plsc.VectorSubcoreMesh(core_axis_name="c", subcore_axis_name="s"),
      scratch_types=[
          pltpu.VMEM((n_tbl,), f32),
          pltpu.VMEM((C,), i32), pltpu.VMEM((C,), i32),
          pltpu.VMEM((C,), i32), pltpu.VMEM((C,), i32),
          pltpu.VMEM((C,), f32), pltpu.VMEM((C,), f32),
          pltpu.VMEM((C,), f32), pltpu.VMEM((C,), f32),
          pltpu.VMEM((C,), i32), pltpu.VMEM((C,), i32),
          pltpu.VMEM((C,), i32), pltpu.VMEM((C,), i32),
          pltpu.VMEM((PROWS, 128), i32),
          pltpu.VMEM((C,), f32),
          pltpu.VMEM_SHARED((n_pad,), f32),
          pltpu.VMEM_SHARED((n_pad,), f32),
          pltpu.SemaphoreType.DMA, pltpu.SemaphoreType.DMA,
          pltpu.SemaphoreType.DMA, pltpu.SemaphoreType.DMA,
          pltpu.SemaphoreType.DMA,
      ],
      compiler_params=pltpu.CompilerParams(needs_layout_passes=False),
  )


def _combine_body(s_ref, c_ref, o_ref):
  sv = s_ref[0] + s_ref[1]
  cv = c_ref[0] + c_ref[1]
  o_ref[...] = sv / jnp.maximum(cv, 1.0)


def kernel(x, w, src, dst, anchors):
  n = x.shape[0]
  t = w.shape[0]
  a = anchors.shape[0]
  n_pad = ((n + 1023) // 1024) * 1024

  e_t = (t // (NW * 128)) * 128
  full = NW * e_t
  tail = t - full
  tailp = ((tail + 127) // 128) * 128

  if tailp:
    padn = tailp - tail
    tsrc = jnp.concatenate([src[full:], jnp.zeros((padn,), jnp.int32)])
    tdst = jnp.concatenate([dst[full:], jnp.full((padn,), n, jnp.int32)])
    tw = jnp.concatenate([w[full:], jnp.zeros((padn,), jnp.float32)])
  else:
    tsrc = jnp.zeros((128,), jnp.int32)
    tdst = jnp.full((128,), n, jnp.int32)
    tw = jnp.zeros((128,), jnp.float32)

  sc_fn = _sc_kernel_fn(n, n_pad, t, a, tailp)
  s_part, c_part, _ = sc_fn(x, w, src, dst, anchors, tsrc, tdst, tw)

  nr = n_pad // 128
  out = pl.pallas_call(
      _combine_body,
      out_shape=jax.ShapeDtypeStruct((nr, 128), jnp.float32),
  )(s_part.reshape(NC, nr, 128), c_part.reshape(NC, nr, 128))

  h_o = out.reshape(n_pad)[:n]
  return (h_o, x)
